# baseline (speedup 1.0000x reference)
import functools

import jax
import jax.numpy as jnp
from jax import lax
from jax.experimental import pallas as pl
from jax.experimental.pallas import tpu as pltpu
from jax.experimental.pallas import tpu_sc as plsc

L = 16
NSUB = 16
NCORE = 2
SUB = 4
SUP = 64


def _round_up(x, m):
    return (x + m - 1) // m * m


def kernel(in_feat, edge_index, W1, b1, W2, b2):
    N, IN_D = in_feat.shape
    HID = W1.shape[1]
    OUT_D = W2.shape[1]
    E = edge_index.shape[1]
    assert E % 128 == 0
    ROWS_E = E // 128

    NROW = _round_up(N + 1, 1024)
    HALF = NROW // 2
    RPT = HALF // NSUB
    assert RPT % 8 == 0
    DUMMY = HALF
    AGG_ROWS = HALF + 8

    edges = edge_index.astype(jnp.int32).reshape(2 * ROWS_E, 1, 128)
    zdeg = jnp.zeros((1, NROW), jnp.float32)
    zrows = jnp.zeros((512, HID), jnp.bfloat16)

    mesh = plsc.VectorSubcoreMesh(core_axis_name="c", subcore_axis_name="s")
    scp = pltpu.CompilerParams(needs_layout_passes=False,
                               use_tc_tiling_on_sc=False)

    @functools.partial(
        pl.kernel,
        out_type=jax.ShapeDtypeStruct((NCORE * NSUB, 1, NROW), jnp.float32),
        mesh=mesh,
        scratch_types=[
            pltpu.VMEM((1, NROW), jnp.float32),
            pltpu.VMEM((16, 1, 128), jnp.int32),
        ],
        compiler_params=scp,
    )
    def deg_kernel(edges_hbm, zdeg_hbm, part_hbm, deg_v, dstb):
        c = lax.axis_index("c")
        s = lax.axis_index("s")
        w = c * NSUB + s
        pltpu.sync_copy(zdeg_hbm, deg_v)
        ones = jnp.full((L,), 1.0, jnp.float32)
        zero16 = jnp.zeros((L,), jnp.int32)

        q, r = divmod(ROWS_E, NCORE * NSUB)
        base = w * q + jnp.minimum(w, r)

        def accum(nrows):
            for j in range(nrows):
                for k in range(128 // L):
                    d = dstb[j, 0, pl.ds(k * L, L)]
                    plsc.addupdate_scatter(deg_v, [zero16, d], ones)

        def do(nrows_w):
            full, tail = divmod(nrows_w, 16)

            def body(i, carry):
                pltpu.sync_copy(
                    edges_hbm.at[pl.ds(ROWS_E + base + i * 16, 16)], dstb)
                accum(16)
                return carry

            lax.fori_loop(0, full, body, 0)
            if tail:
                pltpu.sync_copy(
                    edges_hbm.at[pl.ds(ROWS_E + base + full * 16, tail)],
                    dstb.at[pl.ds(0, tail)])
                accum(tail)

        if r:
            @pl.when(w < r)
            def _():
                do(q + 1)

            @pl.when(w >= r)
            def _():
                do(q)
        else:
            do(q)

        pltpu.sync_copy(deg_v, part_hbm.at[w])

    partials = deg_kernel(edges, zdeg)

    BR = 1024
    grid_a = (pl.cdiv(N, BR),)

    def tca_body(x_ref, w1_ref, b1_ref, part_ref, ones_ref,
                 feat_ref, h_ref, dinv_ref):
        part = part_ref[...].reshape(NCORE * NSUB, BR)
        deg = lax.dot_general(part, ones_ref[...],
                              (((0,), (0,)), ((), ())),
                              preferred_element_type=jnp.float32)
        dinv = lax.rsqrt(jnp.maximum(deg, 1.0))
        f = jnp.dot(x_ref[...], w1_ref[...],
                    preferred_element_type=jnp.float32) + b1_ref[...]
        f = jnp.maximum(f, 0.0)
        feat_ref[...] = f
        h_ref[...] = f * dinv
        dinv_ref[...] = dinv

    feat, h, dinv = pl.pallas_call(
        tca_body,
        grid=grid_a,
        in_specs=[
            pl.BlockSpec((BR, IN_D), lambda i: (i, 0)),
            pl.BlockSpec((IN_D, HID), lambda i: (0, 0)),
            pl.BlockSpec((1, HID), lambda i: (0, 0)),
            pl.BlockSpec((NCORE * NSUB, 1, BR), lambda i: (0, 0, i)),
            pl.BlockSpec((NCORE * NSUB, 1), lambda i: (0, 0)),
        ],
        out_specs=[
            pl.BlockSpec((BR, HID), lambda i: (i, 0)),
            pl.BlockSpec((BR, HID), lambda i: (i, 0)),
            pl.BlockSpec((BR, 1), lambda i: (i, 0)),
        ],
        out_shape=[
            jax.ShapeDtypeStruct((N, HID), jnp.float32),
            jax.ShapeDtypeStruct((N, HID), jnp.float32),
            jax.ShapeDtypeStruct((N, 1), jnp.float32),
        ],
    )(in_feat, W1, b1.reshape(1, -1), partials,
      jnp.ones((NCORE * NSUB, 1), jnp.float32))

    h = h.astype(jnp.bfloat16)

    @functools.partial(
        pl.kernel,
        out_type=jax.ShapeDtypeStruct((NROW, HID), jnp.bfloat16),
        mesh=mesh,
        scratch_types=[
            pltpu.VMEM((SUB * 128, HID), jnp.bfloat16),
            pltpu.VMEM((SUB * 128, HID), jnp.bfloat16),
            pltpu.VMEM((SUP, 1, 128), jnp.int32),
            pltpu.VMEM((SUP, 1, 128), jnp.int32),
            pltpu.VMEM_SHARED((AGG_ROWS, HID), jnp.bfloat16),
            pltpu.SemaphoreType.DMA,
            pltpu.SemaphoreType.DMA,
            pltpu.SemaphoreType.DMA,
            pltpu.SemaphoreType.DMA,
        ],
        compiler_params=scp,
    )
    def seg_kernel(edges_hbm, h_hbm, zrows_hbm, agg_hbm,
                   rows0, rows1, srcb, dstb, agg_sh,
                   gsem0, gsem1, ssem0, ssem1):
        c = lax.axis_index("c")
        s = lax.axis_index("s")
        base = (c * HALF).astype(jnp.int32)
        rows_bufs = (rows0, rows1)
        gsems = (gsem0, gsem1)
        ssems = (ssem0, ssem1)

        pltpu.sync_copy(zrows_hbm, rows0)
        row0 = s * RPT
        for k in range(RPT // 512):
            pltpu.sync_copy(rows0, agg_sh.at[pl.ds(row0 + k * 512, 512)])
        rem = RPT % 512
        if rem:
            pltpu.sync_copy(rows0.at[pl.ds(0, rem)],
                            agg_sh.at[pl.ds(row0 + (RPT // 512) * 512, rem)])
        plsc.subcore_barrier()

        def rebase(nrows, first):
            for j in range(first, first + nrows):
                for k in range(128 // L):
                    v = dstb[j, 0, pl.ds(k * L, L)]
                    ld = v - base
                    ok = (ld >= 0) & (ld < HALF)
                    dstb[j, 0, pl.ds(k * L, L)] = jnp.where(
                        ok, ld, DUMMY + (v & 7))

        def gather(nrows, first, buf):
            return [
                pltpu.async_copy(h_hbm.at[srcb.at[first + j, 0]],
                                 rows_bufs[buf].at[pl.ds(j * 128, 128)],
                                 gsems[buf])
                for j in range(nrows)
            ]

        def scatter(nrows, first, buf):
            return [
                pltpu.async_copy(rows_bufs[buf].at[pl.ds(j * 128, 128)],
                                 agg_sh.at[dstb.at[first + j, 0]],
                                 ssems[buf], add=True)
                for j in range(nrows)
            ]

        q, r = divmod(ROWS_E, NSUB)
        tbase = s * q + jnp.minimum(s, r)

        def super_body(i, carry):
            r0 = tbase + i * SUP
            pltpu.sync_copy(edges_hbm.at[pl.ds(r0, SUP)], srcb)
            pltpu.sync_copy(edges_hbm.at[pl.ds(ROWS_E + r0, SUP)], dstb)
            nsc = SUP // SUB
            g = [None] * nsc
            sc = [None] * nsc
            g[0] = gather(SUB, 0, 0)
            for b in range(nsc):
                buf = b % 2
                if b + 1 < nsc:
                    if b >= 1:
                        for d in sc[b - 1]:
                            d.wait()
                    g[b + 1] = gather(SUB, (b + 1) * SUB, (b + 1) % 2)
                rebase(SUB, b * SUB)
                for d in g[b]:
                    d.wait()
                sc[b] = scatter(SUB, b * SUB, buf)
            for d in sc[nsc - 2] + sc[nsc - 1]:
                d.wait()
            return carry

        def tail_block(r0, nrows):
            pltpu.sync_copy(edges_hbm.at[pl.ds(r0, nrows)],
                            srcb.at[pl.ds(0, nrows)])
            pltpu.sync_copy(edges_hbm.at[pl.ds(ROWS_E + r0, nrows)],
                            dstb.at[pl.ds(0, nrows)])
            g = gather(nrows, 0, 0)
            rebase(nrows, 0)
            for d in g:
                d.wait()
            for d in scatter(nrows, 0, 0):
                d.wait()

        def do(nrows_t):
            full, tail = divmod(nrows_t, SUP)
            lax.fori_loop(0, full, super_body, 0)
            off = full * SUP
            while tail > 0:
                blk = min(tail, SUB)
                tail_block(tbase + off, blk)
                off += blk
                tail -= blk

        if r:
            @pl.when(s < r)
            def _():
                do(q + 1)

            @pl.when(s >= r)
            def _():
                do(q)
        else:
            do(q)

        plsc.subcore_barrier()
        pltpu.sync_copy(agg_sh.at[pl.ds(s * RPT, RPT)],
                        agg_hbm.at[pl.ds(c * HALF + s * RPT, RPT)])

    agg = seg_kernel(edges, h, zrows).astype(jnp.float32)

    def tcb_body(f_ref, a_ref, d_ref, w2_ref, b2_ref, o_ref):
        t = f_ref[...] - a_ref[...] * d_ref[...]
        o_ref[...] = jnp.dot(t, w2_ref[...],
                             preferred_element_type=jnp.float32) + b2_ref[...]

    out = pl.pallas_call(
        tcb_body,
        grid=grid_a,
        in_specs=[
            pl.BlockSpec((BR, HID), lambda i: (i, 0)),
            pl.BlockSpec((BR, HID), lambda i: (i, 0)),
            pl.BlockSpec((BR, 1), lambda i: (i, 0)),
            pl.BlockSpec((HID, OUT_D), lambda i: (0, 0)),
            pl.BlockSpec((1, OUT_D), lambda i: (0, 0)),
        ],
        out_specs=pl.BlockSpec((BR, OUT_D), lambda i: (i, 0)),
        out_shape=jax.ShapeDtypeStruct((N, OUT_D), jnp.float32),
    )(feat, agg, dinv, W2, b2.reshape(1, -1))

    return out


# speedup vs baseline: 1.1539x; 1.1539x over previous
import functools

import jax
import jax.numpy as jnp
from jax import lax
from jax.experimental import pallas as pl
from jax.experimental.pallas import tpu as pltpu
from jax.experimental.pallas import tpu_sc as plsc

L = 16
NSUB = 16
NCORE = 2
SUB = 4
SUP = 64


def _round_up(x, m):
    return (x + m - 1) // m * m


def kernel(in_feat, edge_index, W1, b1, W2, b2):
    N, IN_D = in_feat.shape
    HID = W1.shape[1]
    OUT_D = W2.shape[1]
    E = edge_index.shape[1]
    assert E % 128 == 0
    ROWS_E = E // 128

    NROW = _round_up(N + 1, 1024)
    HHID = HID // 2
    RPT = NROW // NSUB
    assert RPT % 8 == 0

    edges = edge_index.astype(jnp.int32).reshape(2 * ROWS_E, 1, 128)
    zdeg = jnp.zeros((1, NROW), jnp.float32)
    zrows = jnp.zeros((SUB * 128, HHID), jnp.bfloat16)

    mesh = plsc.VectorSubcoreMesh(core_axis_name="c", subcore_axis_name="s")
    scp = pltpu.CompilerParams(needs_layout_passes=False,
                               use_tc_tiling_on_sc=False)

    @functools.partial(
        pl.kernel,
        out_type=jax.ShapeDtypeStruct((NCORE * NSUB, 1, NROW), jnp.float32),
        mesh=mesh,
        scratch_types=[
            pltpu.VMEM((1, NROW), jnp.float32),
            pltpu.VMEM((16, 1, 128), jnp.int32),
        ],
        compiler_params=scp,
    )
    def deg_kernel(edges_hbm, zdeg_hbm, part_hbm, deg_v, dstb):
        c = lax.axis_index("c")
        s = lax.axis_index("s")
        w = c * NSUB + s
        pltpu.sync_copy(zdeg_hbm, deg_v)
        ones = jnp.full((L,), 1.0, jnp.float32)
        zero16 = jnp.zeros((L,), jnp.int32)

        q, r = divmod(ROWS_E, NCORE * NSUB)
        base = w * q + jnp.minimum(w, r)

        def accum(nrows):
            for j in range(nrows):
                for k in range(128 // L):
                    d = dstb[j, 0, pl.ds(k * L, L)]
                    plsc.addupdate_scatter(deg_v, [zero16, d], ones)

        def do(nrows_w):
            full, tail = divmod(nrows_w, 16)

            def body(i, carry):
                pltpu.sync_copy(
                    edges_hbm.at[pl.ds(ROWS_E + base + i * 16, 16)], dstb)
                accum(16)
                return carry

            lax.fori_loop(0, full, body, 0)
            if tail:
                pltpu.sync_copy(
                    edges_hbm.at[pl.ds(ROWS_E + base + full * 16, tail)],
                    dstb.at[pl.ds(0, tail)])
                accum(tail)

        if r:
            @pl.when(w < r)
            def _():
                do(q + 1)

            @pl.when(w >= r)
            def _():
                do(q)
        else:
            do(q)

        pltpu.sync_copy(deg_v, part_hbm.at[w])

    partials = deg_kernel(edges, zdeg)

    BR = 1024
    grid_a = (pl.cdiv(N, BR),)

    def tca_body(x_ref, w1_ref, b1_ref, part_ref, ones_ref,
                 feat_ref, h_ref, dinv_ref):
        part = part_ref[...].reshape(NCORE * NSUB, BR)
        deg = lax.dot_general(part, ones_ref[...],
                              (((0,), (0,)), ((), ())),
                              preferred_element_type=jnp.float32)
        dinv = lax.rsqrt(jnp.maximum(deg, 1.0))
        f = jnp.dot(x_ref[...], w1_ref[...],
                    preferred_element_type=jnp.float32) + b1_ref[...]
        f = jnp.maximum(f, 0.0)
        feat_ref[...] = f
        h_ref[...] = (f * dinv).astype(jnp.bfloat16)
        dinv_ref[...] = dinv

    feat, h, dinv = pl.pallas_call(
        tca_body,
        grid=grid_a,
        in_specs=[
            pl.BlockSpec((BR, IN_D), lambda i: (i, 0)),
            pl.BlockSpec((IN_D, HID), lambda i: (0, 0)),
            pl.BlockSpec((1, HID), lambda i: (0, 0)),
            pl.BlockSpec((NCORE * NSUB, 1, BR), lambda i: (0, 0, i)),
            pl.BlockSpec((NCORE * NSUB, 1), lambda i: (0, 0)),
        ],
        out_specs=[
            pl.BlockSpec((BR, HID), lambda i: (i, 0)),
            pl.BlockSpec((BR, HID), lambda i: (i, 0)),
            pl.BlockSpec((BR, 1), lambda i: (i, 0)),
        ],
        out_shape=[
            jax.ShapeDtypeStruct((N, HID), jnp.float32),
            jax.ShapeDtypeStruct((N, HID), jnp.bfloat16),
            jax.ShapeDtypeStruct((N, 1), jnp.float32),
        ],
    )(in_feat, W1, b1.reshape(1, -1), partials,
      jnp.ones((NCORE * NSUB, 1), jnp.float32))

    @functools.partial(
        pl.kernel,
        out_type=jax.ShapeDtypeStruct((NROW, HID), jnp.bfloat16),
        mesh=mesh,
        scratch_types=[
            pltpu.VMEM((SUB * 128, HHID), jnp.bfloat16),
            pltpu.VMEM((SUB * 128, HHID), jnp.bfloat16),
            pltpu.VMEM((SUP, 1, 128), jnp.int32),
            pltpu.VMEM((SUP, 1, 128), jnp.int32),
            pltpu.VMEM_SHARED((NROW, HHID), jnp.bfloat16),
            pltpu.SemaphoreType.DMA,
            pltpu.SemaphoreType.DMA,
            pltpu.SemaphoreType.DMA,
            pltpu.SemaphoreType.DMA,
        ],
        compiler_params=scp,
    )
    def seg_kernel(edges_hbm, h_hbm, zrows_hbm, agg_hbm,
                   rows0, rows1, srcb, dstb, agg_sh,
                   gsem0, gsem1, ssem0, ssem1):
        c = lax.axis_index("c")
        s = lax.axis_index("s")
        col = c * HHID
        rows_bufs = (rows0, rows1)
        gsems = (gsem0, gsem1)
        ssems = (ssem0, ssem1)

        pltpu.sync_copy(zrows_hbm, rows0)
        row0 = s * RPT
        zc = SUB * 128
        for k in range(RPT // zc):
            pltpu.sync_copy(rows0, agg_sh.at[pl.ds(row0 + k * zc, zc)])
        rem = RPT % zc
        if rem:
            pltpu.sync_copy(rows0.at[pl.ds(0, rem)],
                            agg_sh.at[pl.ds(row0 + (RPT // zc) * zc, rem)])
        plsc.subcore_barrier()

        def scale_src(nrows):
            for j in range(nrows):
                for k in range(128 // L):
                    v = srcb[j, 0, pl.ds(k * L, L)]
                    srcb[j, 0, pl.ds(k * L, L)] = v + v + c

        def gather(nrows, first, buf):
            return [
                pltpu.async_copy(
                    h_hbm.at[srcb.at[first + j, 0]],
                    rows_bufs[buf].at[pl.ds(j * 128, 128)],
                    gsems[buf])
                for j in range(nrows)
            ]

        def scatter(nrows, first, buf):
            return [
                pltpu.async_copy(rows_bufs[buf].at[pl.ds(j * 128, 128)],
                                 agg_sh.at[dstb.at[first + j, 0]],
                                 ssems[buf], add=True)
                for j in range(nrows)
            ]

        q, r = divmod(ROWS_E, NSUB)
        tbase = s * q + jnp.minimum(s, r)

        def super_body(i, carry):
            r0 = tbase + i * SUP
            pltpu.sync_copy(edges_hbm.at[pl.ds(r0, SUP)], srcb)
            pltpu.sync_copy(edges_hbm.at[pl.ds(ROWS_E + r0, SUP)], dstb)
            scale_src(SUP)
            nsc = SUP // SUB
            g = [None] * nsc
            sc = [None] * nsc
            g[0] = gather(SUB, 0, 0)
            for b in range(nsc):
                buf = b % 2
                if b + 1 < nsc:
                    if b >= 1:
                        for d in sc[b - 1]:
                            d.wait()
                    g[b + 1] = gather(SUB, (b + 1) * SUB, (b + 1) % 2)
                for d in g[b]:
                    d.wait()
                sc[b] = scatter(SUB, b * SUB, buf)
            for d in sc[nsc - 2] + sc[nsc - 1]:
                d.wait()
            return carry

        def tail_block(r0, nrows):
            pltpu.sync_copy(edges_hbm.at[pl.ds(r0, nrows)],
                            srcb.at[pl.ds(0, nrows)])
            pltpu.sync_copy(edges_hbm.at[pl.ds(ROWS_E + r0, nrows)],
                            dstb.at[pl.ds(0, nrows)])
            scale_src(nrows)
            g = gather(nrows, 0, 0)
            for d in g:
                d.wait()
            for d in scatter(nrows, 0, 0):
                d.wait()

        def do(nrows_t):
            full, tail = divmod(nrows_t, SUP)
            lax.fori_loop(0, full, super_body, 0)
            off = full * SUP
            while tail > 0:
                blk = min(tail, SUB)
                tail_block(tbase + off, blk)
                off += blk
                tail -= blk

        if r:
            @pl.when(s < r)
            def _():
                do(q + 1)

            @pl.when(s >= r)
            def _():
                do(q)
        else:
            do(q)

        plsc.subcore_barrier()
        pltpu.sync_copy(agg_sh.at[pl.ds(s * RPT, RPT)],
                        agg_hbm.at[pl.ds(s * RPT, RPT), pl.ds(col, HHID)])

    agg = seg_kernel(edges, h.reshape(2 * N, HHID), zrows)

    def tcb_body(f_ref, a_ref, d_ref, w2_ref, b2_ref, o_ref):
        t = f_ref[...] - a_ref[...].astype(jnp.float32) * d_ref[...]
        o_ref[...] = jnp.dot(t, w2_ref[...],
                             preferred_element_type=jnp.float32) + b2_ref[...]

    out = pl.pallas_call(
        tcb_body,
        grid=grid_a,
        in_specs=[
            pl.BlockSpec((BR, HID), lambda i: (i, 0)),
            pl.BlockSpec((BR, HID), lambda i: (i, 0)),
            pl.BlockSpec((BR, 1), lambda i: (i, 0)),
            pl.BlockSpec((HID, OUT_D), lambda i: (0, 0)),
            pl.BlockSpec((1, OUT_D), lambda i: (0, 0)),
        ],
        out_specs=pl.BlockSpec((BR, OUT_D), lambda i: (i, 0)),
        out_shape=jax.ShapeDtypeStruct((N, OUT_D), jnp.float32),
    )(feat, agg, dinv, W2, b2.reshape(1, -1))

    return out


# speedup vs baseline: 1.1975x; 1.0377x over previous
import functools

import jax
import jax.numpy as jnp
from jax import lax
from jax.experimental import pallas as pl
from jax.experimental.pallas import tpu as pltpu
from jax.experimental.pallas import tpu_sc as plsc

L = 16
NSUB = 16
NCORE = 2
SUB = 8
SUP = 64


def _round_up(x, m):
    return (x + m - 1) // m * m


def kernel(in_feat, edge_index, W1, b1, W2, b2):
    N, IN_D = in_feat.shape
    HID = W1.shape[1]
    OUT_D = W2.shape[1]
    E = edge_index.shape[1]
    assert E % 128 == 0
    ROWS_E = E // 128

    NROW = _round_up(N + 1, 1024)
    HHID = HID // 2
    RPT = NROW // NSUB
    assert RPT % 8 == 0

    edges = edge_index.astype(jnp.int32).reshape(2 * ROWS_E, 1, 128)
    zdeg = jnp.zeros((1, NROW), jnp.float32)
    zrows = jnp.zeros((SUB * 128, HHID), jnp.bfloat16)

    mesh = plsc.VectorSubcoreMesh(core_axis_name="c", subcore_axis_name="s")
    scp = pltpu.CompilerParams(needs_layout_passes=False,
                               use_tc_tiling_on_sc=False)

    @functools.partial(
        pl.kernel,
        out_type=jax.ShapeDtypeStruct((NCORE * NSUB, 1, NROW), jnp.float32),
        mesh=mesh,
        scratch_types=[
            pltpu.VMEM((1, NROW), jnp.float32),
            pltpu.VMEM((16, 1, 128), jnp.int32),
        ],
        compiler_params=scp,
    )
    def deg_kernel(edges_hbm, zdeg_hbm, part_hbm, deg_v, dstb):
        c = lax.axis_index("c")
        s = lax.axis_index("s")
        w = c * NSUB + s
        pltpu.sync_copy(zdeg_hbm, deg_v)
        ones = jnp.full((L,), 1.0, jnp.float32)
        zero16 = jnp.zeros((L,), jnp.int32)

        q, r = divmod(ROWS_E, NCORE * NSUB)
        base = w * q + jnp.minimum(w, r)

        def accum(nrows):
            for j in range(nrows):
                for k in range(128 // L):
                    d = dstb[j, 0, pl.ds(k * L, L)]
                    plsc.addupdate_scatter(deg_v, [zero16, d], ones)

        def do(nrows_w):
            full, tail = divmod(nrows_w, 16)

            def body(i, carry):
                pltpu.sync_copy(
                    edges_hbm.at[pl.ds(ROWS_E + base + i * 16, 16)], dstb)
                accum(16)
                return carry

            lax.fori_loop(0, full, body, 0)
            if tail:
                pltpu.sync_copy(
                    edges_hbm.at[pl.ds(ROWS_E + base + full * 16, tail)],
                    dstb.at[pl.ds(0, tail)])
                accum(tail)

        if r:
            @pl.when(w < r)
            def _():
                do(q + 1)

            @pl.when(w >= r)
            def _():
                do(q)
        else:
            do(q)

        pltpu.sync_copy(deg_v, part_hbm.at[w])

    partials = deg_kernel(edges, zdeg)

    BR = 1024
    grid_a = (pl.cdiv(N, BR),)

    def tca_body(x_ref, w1_ref, b1_ref, part_ref, ones_ref,
                 feat_ref, h_ref, dinv_ref):
        part = part_ref[...].reshape(NCORE * NSUB, BR)
        deg = lax.dot_general(part, ones_ref[...],
                              (((0,), (0,)), ((), ())),
                              preferred_element_type=jnp.float32)
        dinv = lax.rsqrt(jnp.maximum(deg, 1.0))
        f = jnp.dot(x_ref[...], w1_ref[...],
                    preferred_element_type=jnp.float32) + b1_ref[...]
        f = jnp.maximum(f, 0.0)
        feat_ref[...] = f
        h_ref[...] = (f * dinv).astype(jnp.bfloat16)
        dinv_ref[...] = dinv

    feat, h, dinv = pl.pallas_call(
        tca_body,
        grid=grid_a,
        in_specs=[
            pl.BlockSpec((BR, IN_D), lambda i: (i, 0)),
            pl.BlockSpec((IN_D, HID), lambda i: (0, 0)),
            pl.BlockSpec((1, HID), lambda i: (0, 0)),
            pl.BlockSpec((NCORE * NSUB, 1, BR), lambda i: (0, 0, i)),
            pl.BlockSpec((NCORE * NSUB, 1), lambda i: (0, 0)),
        ],
        out_specs=[
            pl.BlockSpec((BR, HID), lambda i: (i, 0)),
            pl.BlockSpec((BR, HID), lambda i: (i, 0)),
            pl.BlockSpec((BR, 1), lambda i: (i, 0)),
        ],
        out_shape=[
            jax.ShapeDtypeStruct((N, HID), jnp.float32),
            jax.ShapeDtypeStruct((N, HID), jnp.bfloat16),
            jax.ShapeDtypeStruct((N, 1), jnp.float32),
        ],
    )(in_feat, W1, b1.reshape(1, -1), partials,
      jnp.ones((NCORE * NSUB, 1), jnp.float32))

    @functools.partial(
        pl.kernel,
        out_type=jax.ShapeDtypeStruct((NROW, HID), jnp.bfloat16),
        mesh=mesh,
        scratch_types=[
            pltpu.VMEM((SUB * 128, HHID), jnp.bfloat16),
            pltpu.VMEM((SUB * 128, HHID), jnp.bfloat16),
            pltpu.VMEM((SUP, 1, 128), jnp.int32),
            pltpu.VMEM((SUP, 1, 128), jnp.int32),
            pltpu.VMEM_SHARED((NROW, HHID), jnp.bfloat16),
            pltpu.SemaphoreType.DMA,
            pltpu.SemaphoreType.DMA,
            pltpu.SemaphoreType.DMA,
            pltpu.SemaphoreType.DMA,
        ],
        compiler_params=scp,
    )
    def seg_kernel(edges_hbm, h_hbm, zrows_hbm, agg_hbm,
                   rows0, rows1, srcb, dstb, agg_sh,
                   gsem0, gsem1, ssem0, ssem1):
        c = lax.axis_index("c")
        s = lax.axis_index("s")
        col = c * HHID
        rows_bufs = (rows0, rows1)
        gsems = (gsem0, gsem1)
        ssems = (ssem0, ssem1)

        pltpu.sync_copy(zrows_hbm, rows0)
        row0 = s * RPT
        zc = SUB * 128
        for k in range(RPT // zc):
            pltpu.sync_copy(rows0, agg_sh.at[pl.ds(row0 + k * zc, zc)])
        rem = RPT % zc
        if rem:
            pltpu.sync_copy(rows0.at[pl.ds(0, rem)],
                            agg_sh.at[pl.ds(row0 + (RPT // zc) * zc, rem)])
        plsc.subcore_barrier()

        def scale_src(nrows):
            for j in range(nrows):
                for k in range(128 // L):
                    v = srcb[j, 0, pl.ds(k * L, L)]
                    srcb[j, 0, pl.ds(k * L, L)] = v + v + c

        def gather(nrows, first, buf):
            return [
                pltpu.async_copy(
                    h_hbm.at[srcb.at[first + j, 0]],
                    rows_bufs[buf].at[pl.ds(j * 128, 128)],
                    gsems[buf])
                for j in range(nrows)
            ]

        def scatter(nrows, first, buf):
            return [
                pltpu.async_copy(rows_bufs[buf].at[pl.ds(j * 128, 128)],
                                 agg_sh.at[dstb.at[first + j, 0]],
                                 ssems[buf], add=True)
                for j in range(nrows)
            ]

        q, r = divmod(ROWS_E, NSUB)
        tbase = s * q + jnp.minimum(s, r)

        def super_body(i, carry):
            r0 = tbase + i * SUP
            pltpu.sync_copy(edges_hbm.at[pl.ds(r0, SUP)], srcb)
            pltpu.sync_copy(edges_hbm.at[pl.ds(ROWS_E + r0, SUP)], dstb)
            scale_src(SUP)
            nsc = SUP // SUB
            g = [None] * nsc
            sc = [None] * nsc
            g[0] = gather(SUB, 0, 0)
            for b in range(nsc):
                buf = b % 2
                if b + 1 < nsc:
                    if b >= 1:
                        for d in sc[b - 1]:
                            d.wait()
                    g[b + 1] = gather(SUB, (b + 1) * SUB, (b + 1) % 2)
                for d in g[b]:
                    d.wait()
                sc[b] = scatter(SUB, b * SUB, buf)
            for d in sc[nsc - 2] + sc[nsc - 1]:
                d.wait()
            return carry

        def tail_block(r0, nrows):
            pltpu.sync_copy(edges_hbm.at[pl.ds(r0, nrows)],
                            srcb.at[pl.ds(0, nrows)])
            pltpu.sync_copy(edges_hbm.at[pl.ds(ROWS_E + r0, nrows)],
                            dstb.at[pl.ds(0, nrows)])
            scale_src(nrows)
            g = gather(nrows, 0, 0)
            for d in g:
                d.wait()
            for d in scatter(nrows, 0, 0):
                d.wait()

        def do(nrows_t):
            full, tail = divmod(nrows_t, SUP)
            lax.fori_loop(0, full, super_body, 0)
            off = full * SUP
            while tail > 0:
                blk = min(tail, SUB)
                tail_block(tbase + off, blk)
                off += blk
                tail -= blk

        if r:
            @pl.when(s < r)
            def _():
                do(q + 1)

            @pl.when(s >= r)
            def _():
                do(q)
        else:
            do(q)

        plsc.subcore_barrier()
        pltpu.sync_copy(agg_sh.at[pl.ds(s * RPT, RPT)],
                        agg_hbm.at[pl.ds(s * RPT, RPT), pl.ds(col, HHID)])

    agg = seg_kernel(edges, h.reshape(2 * N, HHID), zrows)

    def tcb_body(f_ref, a_ref, d_ref, w2_ref, b2_ref, o_ref):
        t = f_ref[...] - a_ref[...].astype(jnp.float32) * d_ref[...]
        o_ref[...] = jnp.dot(t, w2_ref[...],
                             preferred_element_type=jnp.float32) + b2_ref[...]

    out = pl.pallas_call(
        tcb_body,
        grid=grid_a,
        in_specs=[
            pl.BlockSpec((BR, HID), lambda i: (i, 0)),
            pl.BlockSpec((BR, HID), lambda i: (i, 0)),
            pl.BlockSpec((BR, 1), lambda i: (i, 0)),
            pl.BlockSpec((HID, OUT_D), lambda i: (0, 0)),
            pl.BlockSpec((1, OUT_D), lambda i: (0, 0)),
        ],
        out_specs=pl.BlockSpec((BR, OUT_D), lambda i: (i, 0)),
        out_shape=jax.ShapeDtypeStruct((N, OUT_D), jnp.float32),
    )(feat, agg, dinv, W2, b2.reshape(1, -1))

    return out


# speedup vs baseline: 1.2211x; 1.0197x over previous
import functools

import jax
import jax.numpy as jnp
from jax import lax
from jax.experimental import pallas as pl
from jax.experimental.pallas import tpu as pltpu
from jax.experimental.pallas import tpu_sc as plsc

L = 16
NSUB = 16
NCORE = 2
SUB = 8
SUP = 64


def _round_up(x, m):
    return (x + m - 1) // m * m


def kernel(in_feat, edge_index, W1, b1, W2, b2):
    N, IN_D = in_feat.shape
    HID = W1.shape[1]
    OUT_D = W2.shape[1]
    E = edge_index.shape[1]
    assert E % 128 == 0
    ROWS_E = E // 128

    NROW = _round_up(N + 1, 1024)
    HHID = HID // 2
    RPT = NROW // NSUB
    assert RPT % 8 == 0

    edges = edge_index.astype(jnp.int32).reshape(2 * ROWS_E, 1, 128)
    zdeg = jnp.zeros((1, NROW), jnp.float32)
    zrows = jnp.zeros((SUB * 128, HHID), jnp.bfloat16)

    mesh = plsc.VectorSubcoreMesh(core_axis_name="c", subcore_axis_name="s")
    scp = pltpu.CompilerParams(needs_layout_passes=False,
                               use_tc_tiling_on_sc=False)

    @functools.partial(
        pl.kernel,
        out_type=jax.ShapeDtypeStruct((NCORE * NSUB, 1, NROW), jnp.float32),
        mesh=mesh,
        scratch_types=[
            pltpu.VMEM((1, NROW), jnp.float32),
            pltpu.VMEM((16, 1, 128), jnp.int32),
            pltpu.VMEM((16, 1, 128), jnp.int32),
            pltpu.SemaphoreType.DMA,
        ],
        compiler_params=scp,
    )
    def deg_kernel(edges_hbm, zdeg_hbm, part_hbm, deg_v, dstb, dstb1, csem):
        c = lax.axis_index("c")
        s = lax.axis_index("s")
        w = c * NSUB + s
        pltpu.sync_copy(zdeg_hbm, deg_v)
        ones = jnp.full((L,), 1.0, jnp.float32)
        zero16 = jnp.zeros((L,), jnp.int32)

        q, r = divmod(ROWS_E, NCORE * NSUB)
        base = w * q + jnp.minimum(w, r)

        def accum(nrows, b):
            for j in range(nrows):
                for k in range(128 // L):
                    d = b[j, 0, pl.ds(k * L, L)]
                    plsc.addupdate_scatter(deg_v, [zero16, d], ones)

        def start(row, b):
            r0 = jnp.minimum(ROWS_E + row, 2 * ROWS_E - 16)
            return pltpu.async_copy(edges_hbm.at[pl.ds(r0, 16)], b, csem)

        def drain(b):
            pltpu.make_async_copy(
                edges_hbm.at[pl.ds(ROWS_E, 16)], b, csem).wait()

        def do(nrows_w):
            full, tail = divmod(nrows_w, 16)
            assert full % 2 == 0
            start(base, dstb)

            def body(g, carry):
                r0 = base + g * 32
                drain(dstb)
                d1 = start(r0 + 16, dstb1)
                accum(16, dstb)
                d1.wait()
                start(r0 + 32, dstb)
                accum(16, dstb1)
                return carry

            lax.fori_loop(0, full // 2, body, 0)
            drain(dstb)
            if tail:
                pltpu.sync_copy(
                    edges_hbm.at[pl.ds(ROWS_E + base + full * 16, tail)],
                    dstb.at[pl.ds(0, tail)])
                accum(tail, dstb)

        if r:
            @pl.when(w < r)
            def _():
                do(q + 1)

            @pl.when(w >= r)
            def _():
                do(q)
        else:
            do(q)

        pltpu.sync_copy(deg_v, part_hbm.at[w])

    partials = deg_kernel(edges, zdeg)

    BR = 1024
    grid_a = (pl.cdiv(N, BR),)

    def tca_body(x_ref, w1_ref, b1_ref, part_ref, ones_ref,
                 feat_ref, h_ref, dinv_ref):
        part = part_ref[...].reshape(NCORE * NSUB, BR)
        deg = lax.dot_general(part, ones_ref[...],
                              (((0,), (0,)), ((), ())),
                              preferred_element_type=jnp.float32)
        dinv = lax.rsqrt(jnp.maximum(deg, 1.0))
        f = jnp.dot(x_ref[...], w1_ref[...],
                    preferred_element_type=jnp.float32) + b1_ref[...]
        f = jnp.maximum(f, 0.0)
        feat_ref[...] = f
        h_ref[...] = (f * dinv).astype(jnp.bfloat16)
        dinv_ref[...] = dinv

    feat, h, dinv = pl.pallas_call(
        tca_body,
        grid=grid_a,
        in_specs=[
            pl.BlockSpec((BR, IN_D), lambda i: (i, 0)),
            pl.BlockSpec((IN_D, HID), lambda i: (0, 0)),
            pl.BlockSpec((1, HID), lambda i: (0, 0)),
            pl.BlockSpec((NCORE * NSUB, 1, BR), lambda i: (0, 0, i)),
            pl.BlockSpec((NCORE * NSUB, 1), lambda i: (0, 0)),
        ],
        out_specs=[
            pl.BlockSpec((BR, HID), lambda i: (i, 0)),
            pl.BlockSpec((BR, HID), lambda i: (i, 0)),
            pl.BlockSpec((BR, 1), lambda i: (i, 0)),
        ],
        out_shape=[
            jax.ShapeDtypeStruct((N, HID), jnp.float32),
            jax.ShapeDtypeStruct((N, HID), jnp.bfloat16),
            jax.ShapeDtypeStruct((N, 1), jnp.float32),
        ],
    )(in_feat, W1, b1.reshape(1, -1), partials,
      jnp.ones((NCORE * NSUB, 1), jnp.float32))

    @functools.partial(
        pl.kernel,
        out_type=jax.ShapeDtypeStruct((NROW, HID), jnp.bfloat16),
        mesh=mesh,
        scratch_types=[
            pltpu.VMEM((SUB * 128, HHID), jnp.bfloat16),
            pltpu.VMEM((SUB * 128, HHID), jnp.bfloat16),
            pltpu.VMEM((SUP, 1, 128), jnp.int32),
            pltpu.VMEM((SUP, 1, 128), jnp.int32),
            pltpu.VMEM_SHARED((NROW, HHID), jnp.bfloat16),
            pltpu.SemaphoreType.DMA,
            pltpu.SemaphoreType.DMA,
            pltpu.SemaphoreType.DMA,
            pltpu.SemaphoreType.DMA,
        ],
        compiler_params=scp,
    )
    def seg_kernel(edges_hbm, h_hbm, zrows_hbm, agg_hbm,
                   rows0, rows1, srcb, dstb, agg_sh,
                   gsem0, gsem1, ssem0, ssem1):
        c = lax.axis_index("c")
        s = lax.axis_index("s")
        col = c * HHID
        rows_bufs = (rows0, rows1)
        gsems = (gsem0, gsem1)
        ssems = (ssem0, ssem1)

        pltpu.sync_copy(zrows_hbm, rows0)
        row0 = s * RPT
        zc = SUB * 128
        for k in range(RPT // zc):
            pltpu.sync_copy(rows0, agg_sh.at[pl.ds(row0 + k * zc, zc)])
        rem = RPT % zc
        if rem:
            pltpu.sync_copy(rows0.at[pl.ds(0, rem)],
                            agg_sh.at[pl.ds(row0 + (RPT // zc) * zc, rem)])
        plsc.subcore_barrier()

        def scale_src(nrows):
            for j in range(nrows):
                for k in range(128 // L):
                    v = srcb[j, 0, pl.ds(k * L, L)]
                    srcb[j, 0, pl.ds(k * L, L)] = v + v + c

        def gather(nrows, first, buf):
            return [
                pltpu.async_copy(
                    h_hbm.at[srcb.at[first + j, 0]],
                    rows_bufs[buf].at[pl.ds(j * 128, 128)],
                    gsems[buf])
                for j in range(nrows)
            ]

        def scatter(nrows, first, buf):
            return [
                pltpu.async_copy(rows_bufs[buf].at[pl.ds(j * 128, 128)],
                                 agg_sh.at[dstb.at[first + j, 0]],
                                 ssems[buf], add=True)
                for j in range(nrows)
            ]

        q, r = divmod(ROWS_E, NSUB)
        tbase = s * q + jnp.minimum(s, r)

        def super_body(i, carry):
            r0 = tbase + i * SUP
            pltpu.sync_copy(edges_hbm.at[pl.ds(r0, SUP)], srcb)
            pltpu.sync_copy(edges_hbm.at[pl.ds(ROWS_E + r0, SUP)], dstb)
            scale_src(SUP)
            nsc = SUP // SUB
            g = [None] * nsc
            sc = [None] * nsc
            g[0] = gather(SUB, 0, 0)
            for b in range(nsc):
                buf = b % 2
                if b + 1 < nsc:
                    if b >= 1:
                        for d in sc[b - 1]:
                            d.wait()
                    g[b + 1] = gather(SUB, (b + 1) * SUB, (b + 1) % 2)
                for d in g[b]:
                    d.wait()
                sc[b] = scatter(SUB, b * SUB, buf)
            for d in sc[nsc - 2] + sc[nsc - 1]:
                d.wait()
            return carry

        def tail_block(r0, nrows):
            pltpu.sync_copy(edges_hbm.at[pl.ds(r0, nrows)],
                            srcb.at[pl.ds(0, nrows)])
            pltpu.sync_copy(edges_hbm.at[pl.ds(ROWS_E + r0, nrows)],
                            dstb.at[pl.ds(0, nrows)])
            scale_src(nrows)
            g = gather(nrows, 0, 0)
            for d in g:
                d.wait()
            for d in scatter(nrows, 0, 0):
                d.wait()

        def do(nrows_t):
            full, tail = divmod(nrows_t, SUP)
            lax.fori_loop(0, full, super_body, 0)
            off = full * SUP
            while tail > 0:
                blk = min(tail, SUB)
                tail_block(tbase + off, blk)
                off += blk
                tail -= blk

        if r:
            @pl.when(s < r)
            def _():
                do(q + 1)

            @pl.when(s >= r)
            def _():
                do(q)
        else:
            do(q)

        plsc.subcore_barrier()
        pltpu.sync_copy(agg_sh.at[pl.ds(s * RPT, RPT)],
                        agg_hbm.at[pl.ds(s * RPT, RPT), pl.ds(col, HHID)])

    agg = seg_kernel(edges, h.reshape(2 * N, HHID), zrows)

    def tcb_body(f_ref, a_ref, d_ref, w2_ref, b2_ref, o_ref):
        t = f_ref[...] - a_ref[...].astype(jnp.float32) * d_ref[...]
        o_ref[...] = jnp.dot(t, w2_ref[...],
                             preferred_element_type=jnp.float32) + b2_ref[...]

    out = pl.pallas_call(
        tcb_body,
        grid=grid_a,
        in_specs=[
            pl.BlockSpec((BR, HID), lambda i: (i, 0)),
            pl.BlockSpec((BR, HID), lambda i: (i, 0)),
            pl.BlockSpec((BR, 1), lambda i: (i, 0)),
            pl.BlockSpec((HID, OUT_D), lambda i: (0, 0)),
            pl.BlockSpec((1, OUT_D), lambda i: (0, 0)),
        ],
        out_specs=pl.BlockSpec((BR, OUT_D), lambda i: (i, 0)),
        out_shape=jax.ShapeDtypeStruct((N, OUT_D), jnp.float32),
    )(feat, agg, dinv, W2, b2.reshape(1, -1))

    return out


# speedup vs baseline: 1.2295x; 1.0068x over previous
import functools

import jax
import jax.numpy as jnp
from jax import lax
from jax.experimental import pallas as pl
from jax.experimental.pallas import tpu as pltpu
from jax.experimental.pallas import tpu_sc as plsc

L = 16
NSUB = 16
NCORE = 2
SUB = 8
SUP = 128


def _round_up(x, m):
    return (x + m - 1) // m * m


def kernel(in_feat, edge_index, W1, b1, W2, b2):
    N, IN_D = in_feat.shape
    HID = W1.shape[1]
    OUT_D = W2.shape[1]
    E = edge_index.shape[1]
    assert E % 128 == 0
    ROWS_E = E // 128

    NROW = _round_up(N + 1, 1024)
    HHID = HID // 2
    RPT = NROW // NSUB
    assert RPT % 8 == 0

    edges = edge_index.astype(jnp.int32).reshape(2 * ROWS_E, 1, 128)
    zdeg = jnp.zeros((1, NROW), jnp.float32)
    zrows = jnp.zeros((SUB * 128, HHID), jnp.bfloat16)

    mesh = plsc.VectorSubcoreMesh(core_axis_name="c", subcore_axis_name="s")
    scp = pltpu.CompilerParams(needs_layout_passes=False,
                               use_tc_tiling_on_sc=False)

    @functools.partial(
        pl.kernel,
        out_type=jax.ShapeDtypeStruct((NCORE * NSUB, 1, NROW), jnp.float32),
        mesh=mesh,
        scratch_types=[
            pltpu.VMEM((1, NROW), jnp.float32),
            pltpu.VMEM((16, 1, 128), jnp.int32),
            pltpu.VMEM((16, 1, 128), jnp.int32),
            pltpu.SemaphoreType.DMA,
        ],
        compiler_params=scp,
    )
    def deg_kernel(edges_hbm, zdeg_hbm, part_hbm, deg_v, dstb, dstb1, csem):
        c = lax.axis_index("c")
        s = lax.axis_index("s")
        w = c * NSUB + s
        pltpu.sync_copy(zdeg_hbm, deg_v)
        ones = jnp.full((L,), 1.0, jnp.float32)
        zero16 = jnp.zeros((L,), jnp.int32)

        q, r = divmod(ROWS_E, NCORE * NSUB)
        base = w * q + jnp.minimum(w, r)

        def accum(nrows, b):
            for j in range(nrows):
                for k in range(128 // L):
                    d = b[j, 0, pl.ds(k * L, L)]
                    plsc.addupdate_scatter(deg_v, [zero16, d], ones)

        def start(row, b):
            r0 = jnp.minimum(ROWS_E + row, 2 * ROWS_E - 16)
            return pltpu.async_copy(edges_hbm.at[pl.ds(r0, 16)], b, csem)

        def drain(b):
            pltpu.make_async_copy(
                edges_hbm.at[pl.ds(ROWS_E, 16)], b, csem).wait()

        def do(nrows_w):
            full, tail = divmod(nrows_w, 16)
            assert full % 2 == 0
            start(base, dstb)

            def body(g, carry):
                r0 = base + g * 32
                drain(dstb)
                d1 = start(r0 + 16, dstb1)
                accum(16, dstb)
                d1.wait()
                start(r0 + 32, dstb)
                accum(16, dstb1)
                return carry

            lax.fori_loop(0, full // 2, body, 0)
            drain(dstb)
            if tail:
                pltpu.sync_copy(
                    edges_hbm.at[pl.ds(ROWS_E + base + full * 16, tail)],
                    dstb.at[pl.ds(0, tail)])
                accum(tail, dstb)

        if r:
            @pl.when(w < r)
            def _():
                do(q + 1)

            @pl.when(w >= r)
            def _():
                do(q)
        else:
            do(q)

        pltpu.sync_copy(deg_v, part_hbm.at[w])

    partials = deg_kernel(edges, zdeg)

    BR = 1024
    grid_a = (pl.cdiv(N, BR),)

    def tca_body(x_ref, w1_ref, b1_ref, part_ref, ones_ref,
                 feat_ref, h_ref, dinv_ref):
        part = part_ref[...].reshape(NCORE * NSUB, BR)
        deg = lax.dot_general(part, ones_ref[...],
                              (((0,), (0,)), ((), ())),
                              preferred_element_type=jnp.float32)
        dinv = lax.rsqrt(jnp.maximum(deg, 1.0))
        f = jnp.dot(x_ref[...], w1_ref[...],
                    preferred_element_type=jnp.float32) + b1_ref[...]
        f = jnp.maximum(f, 0.0)
        feat_ref[...] = f
        h_ref[...] = (f * dinv).astype(jnp.bfloat16)
        dinv_ref[...] = dinv

    feat, h, dinv = pl.pallas_call(
        tca_body,
        grid=grid_a,
        in_specs=[
            pl.BlockSpec((BR, IN_D), lambda i: (i, 0)),
            pl.BlockSpec((IN_D, HID), lambda i: (0, 0)),
            pl.BlockSpec((1, HID), lambda i: (0, 0)),
            pl.BlockSpec((NCORE * NSUB, 1, BR), lambda i: (0, 0, i)),
            pl.BlockSpec((NCORE * NSUB, 1), lambda i: (0, 0)),
        ],
        out_specs=[
            pl.BlockSpec((BR, HID), lambda i: (i, 0)),
            pl.BlockSpec((BR, HID), lambda i: (i, 0)),
            pl.BlockSpec((BR, 1), lambda i: (i, 0)),
        ],
        out_shape=[
            jax.ShapeDtypeStruct((N, HID), jnp.float32),
            jax.ShapeDtypeStruct((N, HID), jnp.bfloat16),
            jax.ShapeDtypeStruct((N, 1), jnp.float32),
        ],
    )(in_feat, W1, b1.reshape(1, -1), partials,
      jnp.ones((NCORE * NSUB, 1), jnp.float32))

    @functools.partial(
        pl.kernel,
        out_type=jax.ShapeDtypeStruct((NROW, HID), jnp.bfloat16),
        mesh=mesh,
        scratch_types=[
            pltpu.VMEM((SUB * 128, HHID), jnp.bfloat16),
            pltpu.VMEM((SUB * 128, HHID), jnp.bfloat16),
            pltpu.VMEM((SUP, 1, 128), jnp.int32),
            pltpu.VMEM((SUP, 1, 128), jnp.int32),
            pltpu.VMEM_SHARED((NROW, HHID), jnp.bfloat16),
            pltpu.SemaphoreType.DMA,
            pltpu.SemaphoreType.DMA,
            pltpu.SemaphoreType.DMA,
            pltpu.SemaphoreType.DMA,
        ],
        compiler_params=scp,
    )
    def seg_kernel(edges_hbm, h_hbm, zrows_hbm, agg_hbm,
                   rows0, rows1, srcb, dstb, agg_sh,
                   gsem0, gsem1, ssem0, ssem1):
        c = lax.axis_index("c")
        s = lax.axis_index("s")
        col = c * HHID
        rows_bufs = (rows0, rows1)
        gsems = (gsem0, gsem1)
        ssems = (ssem0, ssem1)

        pltpu.sync_copy(zrows_hbm, rows0)
        row0 = s * RPT
        zc = SUB * 128
        for k in range(RPT // zc):
            pltpu.sync_copy(rows0, agg_sh.at[pl.ds(row0 + k * zc, zc)])
        rem = RPT % zc
        if rem:
            pltpu.sync_copy(rows0.at[pl.ds(0, rem)],
                            agg_sh.at[pl.ds(row0 + (RPT // zc) * zc, rem)])
        plsc.subcore_barrier()

        def scale_src(nrows):
            for j in range(nrows):
                for k in range(128 // L):
                    v = srcb[j, 0, pl.ds(k * L, L)]
                    srcb[j, 0, pl.ds(k * L, L)] = v + v + c

        def gather(nrows, first, buf):
            return [
                pltpu.async_copy(
                    h_hbm.at[srcb.at[first + j, 0]],
                    rows_bufs[buf].at[pl.ds(j * 128, 128)],
                    gsems[buf])
                for j in range(nrows)
            ]

        def scatter(nrows, first, buf):
            return [
                pltpu.async_copy(rows_bufs[buf].at[pl.ds(j * 128, 128)],
                                 agg_sh.at[dstb.at[first + j, 0]],
                                 ssems[buf], add=True)
                for j in range(nrows)
            ]

        q, r = divmod(ROWS_E, NSUB)
        tbase = s * q + jnp.minimum(s, r)

        def super_body(i, carry):
            r0 = tbase + i * SUP
            pltpu.sync_copy(edges_hbm.at[pl.ds(r0, SUP)], srcb)
            pltpu.sync_copy(edges_hbm.at[pl.ds(ROWS_E + r0, SUP)], dstb)
            scale_src(SUP)
            nsc = SUP // SUB
            g = [None] * nsc
            sc = [None] * nsc
            g[0] = gather(SUB, 0, 0)
            for b in range(nsc):
                buf = b % 2
                if b + 1 < nsc:
                    if b >= 1:
                        for d in sc[b - 1]:
                            d.wait()
                    g[b + 1] = gather(SUB, (b + 1) * SUB, (b + 1) % 2)
                for d in g[b]:
                    d.wait()
                sc[b] = scatter(SUB, b * SUB, buf)
            for d in sc[nsc - 2] + sc[nsc - 1]:
                d.wait()
            return carry

        def tail_block(r0, nrows):
            pltpu.sync_copy(edges_hbm.at[pl.ds(r0, nrows)],
                            srcb.at[pl.ds(0, nrows)])
            pltpu.sync_copy(edges_hbm.at[pl.ds(ROWS_E + r0, nrows)],
                            dstb.at[pl.ds(0, nrows)])
            scale_src(nrows)
            g = gather(nrows, 0, 0)
            for d in g:
                d.wait()
            for d in scatter(nrows, 0, 0):
                d.wait()

        def do(nrows_t):
            full, tail = divmod(nrows_t, SUP)
            lax.fori_loop(0, full, super_body, 0)
            off = full * SUP
            while tail > 0:
                blk = min(tail, SUB)
                tail_block(tbase + off, blk)
                off += blk
                tail -= blk

        if r:
            @pl.when(s < r)
            def _():
                do(q + 1)

            @pl.when(s >= r)
            def _():
                do(q)
        else:
            do(q)

        plsc.subcore_barrier()
        pltpu.sync_copy(agg_sh.at[pl.ds(s * RPT, RPT)],
                        agg_hbm.at[pl.ds(s * RPT, RPT), pl.ds(col, HHID)])

    agg = seg_kernel(edges, h.reshape(2 * N, HHID), zrows)

    def tcb_body(f_ref, a_ref, d_ref, w2_ref, b2_ref, o_ref):
        t = f_ref[...] - a_ref[...].astype(jnp.float32) * d_ref[...]
        o_ref[...] = jnp.dot(t, w2_ref[...],
                             preferred_element_type=jnp.float32) + b2_ref[...]

    out = pl.pallas_call(
        tcb_body,
        grid=grid_a,
        in_specs=[
            pl.BlockSpec((BR, HID), lambda i: (i, 0)),
            pl.BlockSpec((BR, HID), lambda i: (i, 0)),
            pl.BlockSpec((BR, 1), lambda i: (i, 0)),
            pl.BlockSpec((HID, OUT_D), lambda i: (0, 0)),
            pl.BlockSpec((1, OUT_D), lambda i: (0, 0)),
        ],
        out_specs=pl.BlockSpec((BR, OUT_D), lambda i: (i, 0)),
        out_shape=jax.ShapeDtypeStruct((N, OUT_D), jnp.float32),
    )(feat, agg, dinv, W2, b2.reshape(1, -1))

    return out
